# baseline (speedup 1.0000x reference)
import functools

import jax
import jax.numpy as jnp
from jax import lax
from jax.experimental import pallas as pl
from jax.experimental.pallas import tpu as pltpu
from jax.experimental.pallas import tpu_sc as plsc

N, D, E = 8192, 1024, 4
NW = 32
SLOTS = 8
SLOT_TOK = N // SLOTS
BL = 256
BPS = SLOT_TOK // BL
GB = 64


def _gating_body(x_ref, noise_ref, wgn_ref, thr_ref, msk_ref, g_ref):
    xf = x_ref[...]
    g8 = jnp.dot(xf, wgn_ref[...], preferred_element_type=jnp.float32)
    clean = g8[:, :E]
    rawn = g8[:, E:2 * E]
    noise_std = jax.nn.softplus(rawn) + 1e-2
    noisy = clean + noise_ref[...] * noise_std
    scores = noisy - thr_ref[...]
    sel = 0.5 * (jnp.sign(scores) + 1.0) * msk_ref[...]
    masked = jnp.where(sel > 0.0, clean, jnp.full_like(clean, -1e9))
    m = jnp.max(masked, axis=-1, keepdims=True)
    ex = jnp.exp(masked - m)
    sm = ex / jnp.sum(ex, axis=-1, keepdims=True)
    gates = sm * sel
    denom = jnp.clip(jnp.sum(gates, axis=-1, keepdims=True), 1e-9, None)
    g_ref[...] = gates / denom


def _gating(x, wgn, thr, msk, noise):
    BT = 1024
    return pl.pallas_call(
        _gating_body,
        grid=(N // BT,),
        in_specs=[
            pl.BlockSpec((BT, D), lambda i: (i, 0)),
            pl.BlockSpec((BT, E), lambda i: (i, 0)),
            pl.BlockSpec((D, 128), lambda i: (0, 0)),
            pl.BlockSpec((1, E), lambda i: (0, 0)),
            pl.BlockSpec((1, E), lambda i: (0, 0)),
        ],
        out_specs=pl.BlockSpec((BT, E), lambda i: (i, 0)),
        out_shape=jax.ShapeDtypeStruct((N, E), jnp.float32),
    )(x, noise, wgn, thr, msk)


ZROW = NW * SLOT_TOK


def _route_body(gt_hbm, x_hbm, counts_hbm, inv_hbm, gc_hbm, xg_hbm,
                g_v, ids_v, inv_v, gc_v, cnt_v, rows_v, sem):
    wid = lax.axis_index("s") * 2 + lax.axis_index("c")
    e = wid // SLOTS
    base_tok = (wid % SLOTS) * SLOT_TOK
    base_row = wid * SLOT_TOK
    pltpu.sync_copy(gt_hbm.at[e, pl.ds(base_tok, SLOT_TOK)], g_v)

    zeros16 = jnp.zeros((16,), jnp.int32)
    ones16 = jnp.full((16,), 1, jnp.int32)
    zrow16 = jnp.full((16,), ZROW, jnp.int32)
    iota16 = lax.iota(jnp.int32, 16)
    zf16 = jnp.zeros((16,), jnp.float32)

    def chunk(i, cnt):
        ids_v[pl.ds(i * 16, 16)] = zeros16
        g = g_v[pl.ds(i * 16, 16)]
        m = g > zf16
        csum = plsc.cumsum(jnp.where(m, ones16, zeros16))
        pos = jnp.full((16,), cnt, jnp.int32) + csum - ones16
        tok = jnp.full((16,), base_tok + i * 16, jnp.int32) + iota16
        plsc.store_scatter(ids_v, [pos], tok, mask=m)
        plsc.store_scatter(gc_v, [pos], g, mask=m)
        inv_v[pl.ds(i * 16, 16)] = jnp.where(
            m, jnp.full((16,), base_row, jnp.int32) + pos, zrow16)
        return cnt + jnp.max(csum)

    cnt = lax.fori_loop(0, SLOT_TOK // 16, chunk, jnp.int32(0))
    cnt_v[...] = jnp.full((16,), cnt, jnp.int32)
    pltpu.sync_copy(cnt_v, counts_hbm.at[wid])
    pltpu.sync_copy(inv_v, inv_hbm.at[e, pl.ds(base_tok, SLOT_TOK)])
    pltpu.sync_copy(gc_v, gc_hbm.at[wid])

    def batch(bi, _):
        idx = ids_v.at[pl.ds(bi * GB, GB)]
        pltpu.async_copy(x_hbm.at[idx], rows_v, sem).wait()
        pltpu.sync_copy(rows_v, xg_hbm.at[pl.ds(base_row + bi * GB, GB)])
        return 0

    lax.fori_loop(0, (cnt + GB - 1) // GB, batch, 0)


def _route(gates_t, x):
    mesh = plsc.VectorSubcoreMesh(core_axis_name="c", subcore_axis_name="s")
    f = pl.kernel(
        _route_body,
        mesh=mesh,
        out_type=[
            jax.ShapeDtypeStruct((NW, 16), jnp.int32),
            jax.ShapeDtypeStruct((E, N), jnp.int32),
            jax.ShapeDtypeStruct((NW, SLOT_TOK), jnp.float32),
            jax.ShapeDtypeStruct((NW * SLOT_TOK, D), jnp.float32),
        ],
        scratch_types=[
            pltpu.VMEM((SLOT_TOK,), jnp.float32),
            pltpu.VMEM((SLOT_TOK,), jnp.int32),
            pltpu.VMEM((SLOT_TOK,), jnp.int32),
            pltpu.VMEM((SLOT_TOK,), jnp.float32),
            pltpu.VMEM((16,), jnp.int32),
            pltpu.VMEM((GB, D), jnp.float32),
            pltpu.SemaphoreType.DMA,
        ],
        compiler_params=pltpu.CompilerParams(needs_layout_passes=False),
    )
    return f(gates_t, x)


def _gmm_body(cnt_ref, xg_ref, w1_ref, b1_ref, w2_ref, b2_ref, gc_ref, y_ref):
    i = pl.program_id(0)
    b = i % BPS
    cnt = cnt_ref[jnp.minimum(i // BPS, NW - 1), 0]

    @pl.when(i == NW * BPS)
    def _():
        y_ref[...] = jnp.zeros((BL, D), jnp.float32)

    @pl.when((i < NW * BPS) & (b * BL < cnt))
    def _():
        xb = xg_ref[...].astype(jnp.bfloat16)
        h = jnp.dot(xb, w1_ref[0], preferred_element_type=jnp.float32)
        h = h + b1_ref[0]
        hb = jnp.maximum(h, 0.0).astype(jnp.bfloat16)
        y = jnp.dot(hb, w2_ref[0], preferred_element_type=jnp.float32)
        y_ref[...] = (y + b2_ref[0]) * gc_ref[0]


def _gmm(counts, xg, w1b, b1, w2b, b2, gc3):
    def act_row(i, c):
        slot = jnp.minimum(i // BPS, NW - 1)
        nact_m1 = jnp.maximum((c[slot, 0] + BL - 1) // BL - 1, 0)
        return slot * BPS + jnp.minimum(i % BPS, nact_m1)

    def idx_x(i, c):
        return (jnp.where(i == NW * BPS, 0, act_row(i, c)), 0)

    def idx_gc(i, c):
        return (jnp.where(i == NW * BPS, 0, act_row(i, c)), 0, 0)

    def idx_y(i, c):
        slot = jnp.minimum(i // BPS, NW - 1)
        nact = (c[slot, 0] + BL - 1) // BL
        row = slot * BPS + jnp.minimum(i % BPS, nact)
        return (jnp.where(i == NW * BPS, NW * BPS, row), 0)

    def idx_w(i, c):
        return (jnp.minimum(i // (SLOTS * BPS), E - 1), 0, 0)

    grid_spec = pltpu.PrefetchScalarGridSpec(
        num_scalar_prefetch=1,
        grid=(NW * BPS + 1,),
        in_specs=[
            pl.BlockSpec((BL, D), idx_x),
            pl.BlockSpec((1, D, D), idx_w),
            pl.BlockSpec((1, 1, D), idx_w),
            pl.BlockSpec((1, D, D), idx_w),
            pl.BlockSpec((1, 1, D), idx_w),
            pl.BlockSpec((1, BL, 1), idx_gc),
        ],
        out_specs=pl.BlockSpec((BL, D), idx_y),
    )
    return pl.pallas_call(
        _gmm_body,
        grid_spec=grid_spec,
        out_shape=jax.ShapeDtypeStruct((NW * SLOT_TOK + BL, D), jnp.float32),
    )(counts, xg, w1b, b1, w2b, b2, gc3)


def _combine_body(y_hbm, inv_hbm, out_hbm, inv_v, rows_v, acc_v, sem):
    wid = lax.axis_index("s") * 2 + lax.axis_index("c")
    tok0 = wid * (N // NW)
    for e in range(E):
        pltpu.sync_copy(inv_hbm.at[e, pl.ds(tok0, N // NW)], inv_v.at[e])

    for c in range(16):
        for e in range(E):
            idx = inv_v.at[e, pl.ds(c * 16, 16)]
            pltpu.async_copy(y_hbm.at[idx], rows_v.at[e], sem).wait()

        def col(j, _):
            for r in range(16):
                a = rows_v[0, r, pl.ds(j * 16, 16)]
                for e in range(1, E):
                    a = a + rows_v[e, r, pl.ds(j * 16, 16)]
                acc_v[r, pl.ds(j * 16, 16)] = a
            return 0

        lax.fori_loop(0, D // 16, col, 0)
        pltpu.sync_copy(acc_v, out_hbm.at[pl.ds(tok0 + c * 16, 16)])


def _combine(y, inv):
    mesh = plsc.VectorSubcoreMesh(core_axis_name="c", subcore_axis_name="s")
    f = pl.kernel(
        _combine_body,
        mesh=mesh,
        out_type=jax.ShapeDtypeStruct((N, D), jnp.float32),
        scratch_types=[
            pltpu.VMEM((E, N // NW), jnp.int32),
            pltpu.VMEM((E, 16, D), jnp.float32),
            pltpu.VMEM((16, D), jnp.float32),
            pltpu.SemaphoreType.DMA,
        ],
        compiler_params=pltpu.CompilerParams(needs_layout_passes=False),
    )
    return f(y, inv)


def kernel(x, w_gate, w_noise, gate_threshold, experts_mask, noise, W1, b1, W2, b2):
    gw = jnp.concatenate([w_gate, w_noise], axis=1)
    wgn = jnp.pad(gw, ((0, 0), (0, 128 - 2 * E)))
    thr = gate_threshold.reshape(1, E)
    msk = experts_mask.reshape(1, E)
    gates = _gating(x, wgn, thr, msk, noise)
    gates_t = gates.T
    counts, inv, gc, xg = _route(gates_t, x)
    gc3 = gc.reshape(NW * BPS, BL, 1)
    y = _gmm(counts, xg, W1.astype(jnp.bfloat16), b1.reshape(E, 1, D),
             W2.astype(jnp.bfloat16), b2.reshape(E, 1, D), gc3)
    return _combine(y, inv)


# speedup vs baseline: 6.0006x; 6.0006x over previous
import functools

import jax
import jax.numpy as jnp
from jax.experimental import pallas as pl


def _fused_body(x_ref, noise_ref, wgn_ref, thr_ref, msk_ref,
                w1_ref, b1_ref, w2c_ref, out_ref, *, E, D):
    xf = x_ref[...]
    g8 = jnp.dot(xf, wgn_ref[...], preferred_element_type=jnp.float32)
    clean = g8[:, :E]
    rawn = g8[:, E:2 * E]
    noise_std = jax.nn.softplus(rawn) + 1e-2
    noisy = clean + noise_ref[...] * noise_std
    scores = noisy - thr_ref[...]
    sel = 0.5 * (jnp.sign(scores) + 1.0) * msk_ref[...]
    masked = jnp.where(sel > 0.0, clean, jnp.full_like(clean, -1e9))
    m = jnp.max(masked, axis=-1, keepdims=True)
    ex = jnp.exp(masked - m)
    sm = ex / jnp.sum(ex, axis=-1, keepdims=True)
    gates = sm * sel
    denom = jnp.clip(jnp.sum(gates, axis=-1, keepdims=True), 1e-9, None)
    gates = gates / denom
    xb = xf.astype(jnp.bfloat16)
    parts = []
    for e in range(E):
        he = jnp.dot(xb, w1_ref[e], preferred_element_type=jnp.float32)
        he = he + b1_ref[e][None, :]
        parts.append((jnp.maximum(he, 0.0) * gates[:, e:e + 1]).astype(jnp.bfloat16))
    parts.append(jnp.pad(gates, ((0, 0), (0, 128 - E))).astype(jnp.bfloat16))
    hg = jnp.concatenate(parts, axis=1)
    out_ref[...] = jnp.dot(hg, w2c_ref[...], preferred_element_type=jnp.float32)


def kernel(x, w_gate, w_noise, gate_threshold, experts_mask, noise, W1, b1, W2, b2):
    N, D = x.shape
    E = w_gate.shape[1]
    BT = 512
    gw = jnp.concatenate([w_gate, w_noise], axis=1)
    wgn = jnp.pad(gw, ((0, 0), (0, 128 - 2 * E)))
    w1b = W1.astype(jnp.bfloat16)
    w2c = jnp.concatenate(
        [W2.reshape(E * D, D), jnp.pad(b2, ((0, 128 - E), (0, 0)))],
        axis=0).astype(jnp.bfloat16)
    thr = gate_threshold.reshape(1, E)
    msk = experts_mask.reshape(1, E)

    grid = (N // BT,)
    body = functools.partial(_fused_body, E=E, D=D)
    return pl.pallas_call(
        body,
        grid=grid,
        in_specs=[
            pl.BlockSpec((BT, D), lambda i: (i, 0)),
            pl.BlockSpec((BT, E), lambda i: (i, 0)),
            pl.BlockSpec((D, 128), lambda i: (0, 0)),
            pl.BlockSpec((1, E), lambda i: (0, 0)),
            pl.BlockSpec((1, E), lambda i: (0, 0)),
            pl.BlockSpec((E, D, D), lambda i: (0, 0, 0)),
            pl.BlockSpec((E, D), lambda i: (0, 0)),
            pl.BlockSpec((E * D + 128, D), lambda i: (0, 0)),
        ],
        out_specs=pl.BlockSpec((BT, D), lambda i: (i, 0)),
        out_shape=jax.ShapeDtypeStruct((N, D), jnp.float32),
    )(x, noise, wgn, thr, msk, w1b, b1, w2c)


# speedup vs baseline: 6.0213x; 1.0034x over previous
import functools

import jax
import jax.numpy as jnp
from jax.experimental import pallas as pl


def _fused_body(x_ref, noise_ref, wgn_ref, thr_ref, msk_ref,
                w1_ref, b1_ref, w2c_ref, out_ref, *, E, D):
    xf = x_ref[...]
    g8 = jnp.dot(xf, wgn_ref[...], preferred_element_type=jnp.float32)
    clean = g8[:, :E]
    rawn = g8[:, E:2 * E]
    noise_std = jax.nn.softplus(rawn) + 1e-2
    noisy = clean + noise_ref[...] * noise_std
    scores = noisy - thr_ref[...]
    sel = 0.5 * (jnp.sign(scores) + 1.0) * msk_ref[...]
    masked = jnp.where(sel > 0.0, clean, jnp.full_like(clean, -1e9))
    m = jnp.max(masked, axis=-1, keepdims=True)
    ex = jnp.exp(masked - m)
    sm = ex / jnp.sum(ex, axis=-1, keepdims=True)
    gates = sm * sel
    denom = jnp.clip(jnp.sum(gates, axis=-1, keepdims=True), 1e-9, None)
    gates = gates / denom
    xb = xf.astype(jnp.bfloat16)
    parts = []
    for e in range(E):
        he = jnp.dot(xb, w1_ref[e], preferred_element_type=jnp.float32)
        he = he + b1_ref[e][None, :]
        parts.append((jnp.maximum(he, 0.0) * gates[:, e:e + 1]).astype(jnp.bfloat16))
    parts.append(jnp.pad(gates, ((0, 0), (0, 128 - E))).astype(jnp.bfloat16))
    hg = jnp.concatenate(parts, axis=1)
    out_ref[...] = jnp.dot(hg, w2c_ref[...], preferred_element_type=jnp.float32)


def kernel(x, w_gate, w_noise, gate_threshold, experts_mask, noise, W1, b1, W2, b2):
    N, D = x.shape
    E = w_gate.shape[1]
    BT = 1024
    gw = jnp.concatenate([w_gate, w_noise], axis=1)
    wgn = jnp.pad(gw, ((0, 0), (0, 128 - 2 * E)))
    w1b = W1.astype(jnp.bfloat16)
    w2c = jnp.concatenate(
        [W2.reshape(E * D, D), jnp.pad(b2, ((0, 128 - E), (0, 0)))],
        axis=0).astype(jnp.bfloat16)
    thr = gate_threshold.reshape(1, E)
    msk = experts_mask.reshape(1, E)

    grid = (N // BT,)
    body = functools.partial(_fused_body, E=E, D=D)
    return pl.pallas_call(
        body,
        grid=grid,
        in_specs=[
            pl.BlockSpec((BT, D), lambda i: (i, 0)),
            pl.BlockSpec((BT, E), lambda i: (i, 0)),
            pl.BlockSpec((D, 128), lambda i: (0, 0)),
            pl.BlockSpec((1, E), lambda i: (0, 0)),
            pl.BlockSpec((1, E), lambda i: (0, 0)),
            pl.BlockSpec((E, D, D), lambda i: (0, 0, 0)),
            pl.BlockSpec((E, D), lambda i: (0, 0)),
            pl.BlockSpec((E * D + 128, D), lambda i: (0, 0)),
        ],
        out_specs=pl.BlockSpec((BT, D), lambda i: (i, 0)),
        out_shape=jax.ShapeDtypeStruct((N, D), jnp.float32),
    )(x, noise, wgn, thr, msk, w1b, b1, w2c)
